# baseline (speedup 1.0000x reference)
import functools

import jax
import jax.numpy as jnp
from jax import lax
from jax.experimental import pallas as pl
from jax.experimental.pallas import tpu as pltpu
from jax.experimental.pallas import tpu_sc as plsc

N = 10000
E = 320000
D = 128
NC = 2
NS = 16
NW = NC * NS
C = 128
NCHUNK = 80
BF = 4
EPT = C * NCHUNK
EPAD = EPT - E // NW
NPAD = 10112
RPS = NPAD // NS

_mesh = plsc.VectorSubcoreMesh(core_axis_name="c", subcore_axis_name="s")


NHALF = NCHUNK // 2
NPAIR = NHALF // 2


@functools.partial(
    pl.kernel,
    out_type=(
        jax.ShapeDtypeStruct((NC, NPAD, D), jnp.float32),
        jax.ShapeDtypeStruct((NC * NPAD,), jnp.float32),
    ),
    mesh=_mesh,
    scratch_types=[
        pltpu.VMEM((NHALF, C), jnp.int32),
        pltpu.VMEM((NHALF, C), jnp.int32),
        pltpu.VMEM((C, D), jnp.float32),
        pltpu.VMEM((C, D), jnp.float32),
        pltpu.VMEM((C,), jnp.float32),
        pltpu.VMEM_SHARED((NPAD, D), jnp.float32),
        pltpu.VMEM_SHARED((NPAD,), jnp.float32),
        pltpu.SemaphoreType.DMA,
        pltpu.SemaphoreType.DMA,
        pltpu.SemaphoreType.DMA,
        pltpu.SemaphoreType.DMA,
        pltpu.SemaphoreType.DMA,
    ],
)
def _sc_aggregate(h_hbm, src_hbm, dst_hbm, zacc_hbm, zcnt_hbm,
                  sum_hbm, cnt_hbm,
                  src_v, dst_v, rows0_v, rows1_v, ones_v, acc_sp, cnt_sp,
                  semg0, semg1, sems0, sems1, semc):
    c = lax.axis_index("c")
    s = lax.axis_index("s")
    wid = s * NC + c

    for k in range(C // 16):
        ones_v[pl.ds(k * 16, 16)] = jnp.full((16,), 1.0, jnp.float32)

    r0 = s * RPS
    pltpu.async_copy(zacc_hbm, acc_sp.at[pl.ds(r0, RPS)], sems0)

    @pl.when(s == 0)
    def _():
        pltpu.async_copy(zcnt_hbm, cnt_sp, sems1)

    pltpu.make_async_copy(zacc_hbm, acc_sp.at[pl.ds(r0, RPS)], sems0).wait()

    @pl.when(s == 0)
    def _():
        pltpu.make_async_copy(zcnt_hbm, cnt_sp, sems1).wait()

    plsc.subcore_barrier()

    for half in range(2):
        pltpu.async_copy(src_hbm.at[wid, half], src_v, semg0)
        pltpu.async_copy(dst_hbm.at[wid, half], dst_v, semg1)
        pltpu.make_async_copy(src_hbm.at[wid, half], src_v, semg0).wait()
        pltpu.make_async_copy(dst_hbm.at[wid, half], dst_v, semg1).wait()
        pltpu.async_copy(h_hbm.at[src_v.at[0]], rows0_v, semg0)

        def body(g, carry):
            a = 2 * g
            b = a + 1

            @pl.when(g > 0)
            def _():
                pltpu.make_async_copy(rows1_v, acc_sp.at[dst_v.at[b]],
                                      sems1).wait()

            pltpu.async_copy(h_hbm.at[src_v.at[b]], rows1_v, semg1)

            pltpu.make_async_copy(h_hbm.at[src_v.at[a]], rows0_v,
                                  semg0).wait()
            pltpu.async_copy(rows0_v, acc_sp.at[dst_v.at[a]], sems0,
                             add=True)
            pltpu.async_copy(ones_v, cnt_sp.at[dst_v.at[a]], semc, add=True)

            @pl.when(g < NPAIR - 1)
            def _():
                pltpu.make_async_copy(rows0_v, acc_sp.at[dst_v.at[a]],
                                      sems0).wait()
                pltpu.async_copy(h_hbm.at[src_v.at[a + 2]], rows0_v, semg0)

            pltpu.make_async_copy(h_hbm.at[src_v.at[b]], rows1_v,
                                  semg1).wait()
            pltpu.async_copy(rows1_v, acc_sp.at[dst_v.at[b]], sems1,
                             add=True)
            pltpu.async_copy(ones_v, cnt_sp.at[dst_v.at[b]], semc, add=True)
            return carry

        lax.fori_loop(0, NPAIR, body, 0)

        pltpu.make_async_copy(rows0_v, acc_sp.at[dst_v.at[0]], sems0).wait()
        pltpu.make_async_copy(rows1_v, acc_sp.at[dst_v.at[0]], sems1).wait()

        pltpu.make_async_copy(rows0_v.at[pl.ds(0, NHALF * C * 4 // (D * 4))],
                              acc_sp.at[pl.ds(0, NHALF * C * 4 // (D * 4))],
                              semc).wait()

    plsc.subcore_barrier()

    pltpu.sync_copy(acc_sp.at[pl.ds(r0, RPS)], sum_hbm.at[c, pl.ds(r0, RPS)])

    @pl.when(s == 0)
    def _():
        pltpu.sync_copy(cnt_sp, cnt_hbm.at[pl.ds(c * NPAD, NPAD)])


BM = 400


def _tc_pre_body(h_ref, w_ref, b_ref, o_ref):
    o_ref[...] = lax.dot_general(
        h_ref[...], w_ref[:, :D], (((1,), (1,)), ((), ())),
        preferred_element_type=jnp.float32) + b_ref[...]


def _tc_pre(h, W, b2):
    return pl.pallas_call(
        _tc_pre_body,
        grid=(N // BM,),
        in_specs=[
            pl.BlockSpec((BM, D), lambda i: (i, 0)),
            pl.BlockSpec((D, 2 * D), lambda i: (0, 0)),
            pl.BlockSpec((1, D), lambda i: (0, 0)),
        ],
        out_specs=pl.BlockSpec((BM, D), lambda i: (i, 0)),
        out_shape=jax.ShapeDtypeStruct((N, D), jnp.float32),
        compiler_params=pltpu.CompilerParams(
            dimension_semantics=("parallel",),
        ),
    )(h, W, b2)


def _tc_body(h_ref, pre_ref, p_ref, cnt_ref, w_ref, o_ref):
    h_b = h_ref[...]
    p = p_ref[0] + p_ref[1]
    cnt = cnt_ref[:, 0:1] + cnt_ref[:, 1:2]
    cmean = p / jnp.maximum(cnt, 1.0)
    z = pre_ref[...] + lax.dot_general(
        cmean, w_ref[:, D:], (((1,), (1,)), ((), ())),
        preferred_element_type=jnp.float32)
    nrm = jnp.sqrt(jnp.sum(z * z, axis=1, keepdims=True))
    z = z / jnp.maximum(nrm, 1e-12)
    o_ref[...] = h_b + jnp.maximum(z, 0.0)


def _tc_combine(h, pre, partial, cnts, W):
    grid = (N // BM,)
    return pl.pallas_call(
        _tc_body,
        grid=grid,
        in_specs=[
            pl.BlockSpec((BM, D), lambda i: (i, 0)),
            pl.BlockSpec((BM, D), lambda i: (i, 0)),
            pl.BlockSpec((NC, BM, D), lambda i: (0, i, 0)),
            pl.BlockSpec((BM, NC), lambda i: (i, 0)),
            pl.BlockSpec((D, 2 * D), lambda i: (0, 0)),
        ],
        out_specs=pl.BlockSpec((BM, D), lambda i: (i, 0)),
        out_shape=jax.ShapeDtypeStruct((N, D), jnp.float32),
        compiler_params=pltpu.CompilerParams(
            dimension_semantics=("parallel",),
        ),
    )(h, pre, partial, cnts, W)


def _pad_indices(idx, fill):
    per = idx.reshape(NW, E // NW)
    pad = jnp.broadcast_to(fill, (NW, EPAD))
    return jnp.concatenate([per, pad], axis=1).reshape(NW, 2, NCHUNK // 2, C)


def kernel(h, edge_index, W, b):
    src_fill = jnp.arange(EPAD, dtype=jnp.int32) % N
    dst_fill = N + jnp.arange(EPAD, dtype=jnp.int32) % (NPAD - N)
    src = _pad_indices(edge_index[0].astype(jnp.int32), src_fill)
    dst = _pad_indices(edge_index[1].astype(jnp.int32), dst_fill)
    zacc = jnp.zeros((RPS, D), jnp.float32)
    zcnt = jnp.zeros((NPAD,), jnp.float32)
    part_sum, cnt_flat = _sc_aggregate(h, src, dst, zacc, zcnt)
    pre = _tc_pre(h, W, b.reshape(1, D))
    cnt_pair = cnt_flat.reshape(NC, NPAD).T
    out = _tc_combine(h, pre, part_sum, cnt_pair, W)
    return out


# speedup vs baseline: 1.0191x; 1.0191x over previous
import functools

import jax
import jax.numpy as jnp
from jax import lax
from jax.experimental import pallas as pl
from jax.experimental.pallas import tpu as pltpu
from jax.experimental.pallas import tpu_sc as plsc

N = 10000
E = 320000
D = 128
NC = 2
NS = 16
NW = NC * NS
C = 128
NCHUNK = 80
BF = 4
EPT = C * NCHUNK
EPAD = EPT - E // NW
NPAD = 10112
RPS = NPAD // NS

_mesh = plsc.VectorSubcoreMesh(core_axis_name="c", subcore_axis_name="s")


NHALF = NCHUNK // 2
NPAIR = NHALF // 2


@functools.partial(
    pl.kernel,
    out_type=(
        jax.ShapeDtypeStruct((NC, NPAD, D), jnp.float32),
        jax.ShapeDtypeStruct((NC * NPAD,), jnp.float32),
    ),
    mesh=_mesh,
    scratch_types=[
        pltpu.VMEM((NHALF, C), jnp.int32),
        pltpu.VMEM((NHALF, C), jnp.int32),
        pltpu.VMEM((C, D), jnp.float32),
        pltpu.VMEM((C, D), jnp.float32),
        pltpu.VMEM((C,), jnp.float32),
        pltpu.VMEM_SHARED((NPAD, D), jnp.float32),
        pltpu.VMEM_SHARED((NPAD,), jnp.float32),
        pltpu.SemaphoreType.DMA,
        pltpu.SemaphoreType.DMA,
        pltpu.SemaphoreType.DMA,
        pltpu.SemaphoreType.DMA,
        pltpu.SemaphoreType.DMA,
    ],
)
def _sc_aggregate(h_hbm, src_hbm, dst_hbm, zacc_hbm, zcnt_hbm,
                  sum_hbm, cnt_hbm,
                  src_v, dst_v, rows0_v, rows1_v, ones_v, acc_sp, cnt_sp,
                  semg0, semg1, sems0, sems1, semc):
    c = lax.axis_index("c")
    s = lax.axis_index("s")
    wid = s * NC + c

    for k in range(C // 16):
        ones_v[pl.ds(k * 16, 16)] = jnp.full((16,), 1.0, jnp.float32)

    r0 = s * RPS
    pltpu.async_copy(zacc_hbm, acc_sp.at[pl.ds(r0, RPS)], sems0)

    @pl.when(s == 0)
    def _():
        pltpu.async_copy(zcnt_hbm, cnt_sp, sems1)

    pltpu.make_async_copy(zacc_hbm, acc_sp.at[pl.ds(r0, RPS)], sems0).wait()

    @pl.when(s == 0)
    def _():
        pltpu.make_async_copy(zcnt_hbm, cnt_sp, sems1).wait()

    plsc.subcore_barrier()

    for half in range(2):
        pltpu.async_copy(src_hbm.at[wid, half], src_v, semg0)
        pltpu.async_copy(dst_hbm.at[wid, half], dst_v, semg1)
        pltpu.make_async_copy(src_hbm.at[wid, half], src_v, semg0).wait()
        pltpu.make_async_copy(dst_hbm.at[wid, half], dst_v, semg1).wait()
        pltpu.async_copy(h_hbm.at[src_v.at[0]], rows0_v, semg0)

        def body(g, carry):
            a = 2 * g
            b = a + 1

            @pl.when(g > 0)
            def _():
                pltpu.make_async_copy(rows1_v, acc_sp.at[dst_v.at[b]],
                                      sems1).wait()

            pltpu.async_copy(h_hbm.at[src_v.at[b]], rows1_v, semg1)

            pltpu.make_async_copy(h_hbm.at[src_v.at[a]], rows0_v,
                                  semg0).wait()
            pltpu.async_copy(rows0_v, acc_sp.at[dst_v.at[a]], sems0,
                             add=True)
            pltpu.async_copy(ones_v, cnt_sp.at[dst_v.at[a]], semc, add=True)

            @pl.when(g < NPAIR - 1)
            def _():
                pltpu.make_async_copy(rows0_v, acc_sp.at[dst_v.at[a]],
                                      sems0).wait()
                pltpu.async_copy(h_hbm.at[src_v.at[a + 2]], rows0_v, semg0)

            pltpu.make_async_copy(h_hbm.at[src_v.at[b]], rows1_v,
                                  semg1).wait()
            pltpu.async_copy(rows1_v, acc_sp.at[dst_v.at[b]], sems1,
                             add=True)
            pltpu.async_copy(ones_v, cnt_sp.at[dst_v.at[b]], semc, add=True)
            return carry

        lax.fori_loop(0, NPAIR, body, 0)

        pltpu.make_async_copy(rows0_v, acc_sp.at[dst_v.at[0]], sems0).wait()
        pltpu.make_async_copy(rows1_v, acc_sp.at[dst_v.at[0]], sems1).wait()

        pltpu.make_async_copy(rows0_v.at[pl.ds(0, NHALF * C * 4 // (D * 4))],
                              acc_sp.at[pl.ds(0, NHALF * C * 4 // (D * 4))],
                              semc).wait()

    plsc.subcore_barrier()

    pltpu.sync_copy(acc_sp.at[pl.ds(r0, RPS)], sum_hbm.at[c, pl.ds(r0, RPS)])

    @pl.when(s == 0)
    def _():
        pltpu.sync_copy(cnt_sp, cnt_hbm.at[pl.ds(c * NPAD, NPAD)])


BM = 400


def _tc_body(h_ref, p_ref, cnt_ref, w_ref, b_ref, o_ref):
    h_b = h_ref[...]
    p = p_ref[0] + p_ref[1]
    cnt = cnt_ref[:, 0:1] + cnt_ref[:, 1:2]
    cmean = p / jnp.maximum(cnt, 1.0)
    z = (
        lax.dot_general(h_b, w_ref[:, :D], (((1,), (1,)), ((), ())),
                        preferred_element_type=jnp.float32)
        + lax.dot_general(cmean, w_ref[:, D:], (((1,), (1,)), ((), ())),
                          preferred_element_type=jnp.float32)
        + b_ref[...]
    )
    nrm = jnp.sqrt(jnp.sum(z * z, axis=1, keepdims=True))
    z = z / jnp.maximum(nrm, 1e-12)
    o_ref[...] = h_b + jnp.maximum(z, 0.0)


def _tc_combine(h, partial, cnts, W, b2):
    grid = (N // BM,)
    return pl.pallas_call(
        _tc_body,
        grid=grid,
        in_specs=[
            pl.BlockSpec((BM, D), lambda i: (i, 0)),
            pl.BlockSpec((NC, BM, D), lambda i: (0, i, 0)),
            pl.BlockSpec((BM, NC), lambda i: (i, 0)),
            pl.BlockSpec((D, 2 * D), lambda i: (0, 0)),
            pl.BlockSpec((1, D), lambda i: (0, 0)),
        ],
        out_specs=pl.BlockSpec((BM, D), lambda i: (i, 0)),
        out_shape=jax.ShapeDtypeStruct((N, D), jnp.float32),
        compiler_params=pltpu.CompilerParams(
            dimension_semantics=("parallel",),
        ),
    )(h, partial, cnts, W, b2)


def _pad_indices(idx, fill):
    per = idx.reshape(NW, E // NW)
    pad = jnp.broadcast_to(fill, (NW, EPAD))
    return jnp.concatenate([per, pad], axis=1).reshape(NW, 2, NCHUNK // 2, C)


def kernel(h, edge_index, W, b):
    src_fill = jnp.arange(EPAD, dtype=jnp.int32) % N
    dst_fill = N + jnp.arange(EPAD, dtype=jnp.int32) % (NPAD - N)
    src = _pad_indices(edge_index[0].astype(jnp.int32), src_fill)
    dst = _pad_indices(edge_index[1].astype(jnp.int32), dst_fill)
    zacc = jnp.zeros((RPS, D), jnp.float32)
    zcnt = jnp.zeros((NPAD,), jnp.float32)
    part_sum, cnt_flat = _sc_aggregate(h, src, dst, zacc, zcnt)
    cnt_pair = cnt_flat.reshape(NC, NPAD).T
    out = _tc_combine(h, part_sum, cnt_pair, W, b.reshape(1, D))
    return out


# speedup vs baseline: 1.0694x; 1.0494x over previous
import functools

import jax
import jax.numpy as jnp
from jax import lax
from jax.experimental import pallas as pl
from jax.experimental.pallas import tpu as pltpu
from jax.experimental.pallas import tpu_sc as plsc

N = 10000
E = 320000
D = 128
NC = 2
NS = 16
NW = NC * NS
C = 128
NCHUNK = 80
BF = 4
EPT = C * NCHUNK
EPAD = EPT - E // NW
NPAD = 10112
RPS = NPAD // NS

_mesh = plsc.VectorSubcoreMesh(core_axis_name="c", subcore_axis_name="s")


NHALF = NCHUNK // 2
NPAIR = NHALF // 2


@functools.partial(
    pl.kernel,
    out_type=(
        jax.ShapeDtypeStruct((NC, NPAD, D), jnp.float32),
        jax.ShapeDtypeStruct((NC * NPAD,), jnp.float32),
    ),
    mesh=_mesh,
    scratch_types=[
        pltpu.VMEM((NHALF, C), jnp.int32),
        pltpu.VMEM((NHALF, C), jnp.int32),
        pltpu.VMEM((C, D), jnp.float32),
        pltpu.VMEM((C, D), jnp.float32),
        pltpu.VMEM((C,), jnp.float32),
        pltpu.VMEM_SHARED((NPAD, D), jnp.float32),
        pltpu.VMEM_SHARED((NPAD,), jnp.float32),
        pltpu.SemaphoreType.DMA,
        pltpu.SemaphoreType.DMA,
        pltpu.SemaphoreType.DMA,
        pltpu.SemaphoreType.DMA,
        pltpu.SemaphoreType.DMA,
    ],
)
def _sc_aggregate(h_hbm, src_hbm, dst_hbm, zacc_hbm, zcnt_hbm,
                  sum_hbm, cnt_hbm,
                  src_v, dst_v, rows0_v, rows1_v, ones_v, acc_sp, cnt_sp,
                  semg0, semg1, sems0, sems1, semc):
    c = lax.axis_index("c")
    s = lax.axis_index("s")
    wid = s * NC + c

    for k in range(C // 16):
        ones_v[pl.ds(k * 16, 16)] = jnp.full((16,), 1.0, jnp.float32)

    r0 = s * RPS
    pltpu.async_copy(zacc_hbm, acc_sp.at[pl.ds(r0, RPS)], sems0)

    @pl.when(s == 0)
    def _():
        pltpu.async_copy(zcnt_hbm, cnt_sp, sems1)

    pltpu.make_async_copy(zacc_hbm, acc_sp.at[pl.ds(r0, RPS)], sems0).wait()

    @pl.when(s == 0)
    def _():
        pltpu.make_async_copy(zcnt_hbm, cnt_sp, sems1).wait()

    plsc.subcore_barrier()

    for half in range(2):
        pltpu.async_copy(src_hbm.at[wid, half], src_v, semg0)
        pltpu.async_copy(dst_hbm.at[wid, half], dst_v, semg1)
        pltpu.make_async_copy(src_hbm.at[wid, half], src_v, semg0).wait()
        pltpu.make_async_copy(dst_hbm.at[wid, half], dst_v, semg1).wait()
        pltpu.async_copy(h_hbm.at[src_v.at[0]], rows0_v, semg0)

        def body(g, carry):
            a = 2 * g
            b = a + 1

            @pl.when(g > 0)
            def _():
                pltpu.make_async_copy(rows1_v, acc_sp.at[dst_v.at[b]],
                                      sems1).wait()

            pltpu.async_copy(h_hbm.at[src_v.at[b]], rows1_v, semg1)

            pltpu.make_async_copy(h_hbm.at[src_v.at[a]], rows0_v,
                                  semg0).wait()
            pltpu.async_copy(rows0_v, acc_sp.at[dst_v.at[a]], sems0,
                             add=True)
            pltpu.async_copy(ones_v, cnt_sp.at[dst_v.at[a]], semc, add=True)

            @pl.when(g < NPAIR - 1)
            def _():
                pltpu.make_async_copy(rows0_v, acc_sp.at[dst_v.at[a]],
                                      sems0).wait()
                pltpu.async_copy(h_hbm.at[src_v.at[a + 2]], rows0_v, semg0)

            pltpu.make_async_copy(h_hbm.at[src_v.at[b]], rows1_v,
                                  semg1).wait()
            pltpu.async_copy(rows1_v, acc_sp.at[dst_v.at[b]], sems1,
                             add=True)
            pltpu.async_copy(ones_v, cnt_sp.at[dst_v.at[b]], semc, add=True)
            return carry

        lax.fori_loop(0, NPAIR, body, 0)

        pltpu.make_async_copy(rows0_v, acc_sp.at[dst_v.at[0]], sems0).wait()
        pltpu.make_async_copy(rows1_v, acc_sp.at[dst_v.at[0]], sems1).wait()

        pltpu.make_async_copy(rows0_v.at[pl.ds(0, NHALF * C * 4 // (D * 4))],
                              acc_sp.at[pl.ds(0, NHALF * C * 4 // (D * 4))],
                              semc).wait()

    plsc.subcore_barrier()

    pltpu.sync_copy(acc_sp.at[pl.ds(r0, RPS)], sum_hbm.at[c, pl.ds(r0, RPS)])

    @pl.when(s == 0)
    def _():
        pltpu.sync_copy(cnt_sp, cnt_hbm.at[pl.ds(c * NPAD, NPAD)])


BM = 1000


def _tc_body(h_ref, p_ref, cnt_ref, w_ref, b_ref, o_ref):
    h_b = h_ref[...]
    p = p_ref[0] + p_ref[1]
    cnt = cnt_ref[:, 0:1] + cnt_ref[:, 1:2]
    cmean = p / jnp.maximum(cnt, 1.0)
    z = (
        lax.dot_general(h_b, w_ref[:, :D], (((1,), (1,)), ((), ())),
                        preferred_element_type=jnp.float32)
        + lax.dot_general(cmean, w_ref[:, D:], (((1,), (1,)), ((), ())),
                          preferred_element_type=jnp.float32)
        + b_ref[...]
    )
    nrm = jnp.sqrt(jnp.sum(z * z, axis=1, keepdims=True))
    z = z / jnp.maximum(nrm, 1e-12)
    o_ref[...] = h_b + jnp.maximum(z, 0.0)


def _tc_combine(h, partial, cnts, W, b2):
    grid = (N // BM,)
    return pl.pallas_call(
        _tc_body,
        grid=grid,
        in_specs=[
            pl.BlockSpec((BM, D), lambda i: (i, 0)),
            pl.BlockSpec((NC, BM, D), lambda i: (0, i, 0)),
            pl.BlockSpec((BM, NC), lambda i: (i, 0)),
            pl.BlockSpec((D, 2 * D), lambda i: (0, 0)),
            pl.BlockSpec((1, D), lambda i: (0, 0)),
        ],
        out_specs=pl.BlockSpec((BM, D), lambda i: (i, 0)),
        out_shape=jax.ShapeDtypeStruct((N, D), jnp.float32),
        compiler_params=pltpu.CompilerParams(
            dimension_semantics=("parallel",),
        ),
    )(h, partial, cnts, W, b2)


def _pad_indices(idx, fill):
    per = idx.reshape(NW, E // NW)
    pad = jnp.broadcast_to(fill, (NW, EPAD))
    return jnp.concatenate([per, pad], axis=1).reshape(NW, 2, NCHUNK // 2, C)


def kernel(h, edge_index, W, b):
    src_fill = jnp.arange(EPAD, dtype=jnp.int32) % N
    dst_fill = N + jnp.arange(EPAD, dtype=jnp.int32) % (NPAD - N)
    src = _pad_indices(edge_index[0].astype(jnp.int32), src_fill)
    dst = _pad_indices(edge_index[1].astype(jnp.int32), dst_fill)
    zacc = jnp.zeros((RPS, D), jnp.float32)
    zcnt = jnp.zeros((NPAD,), jnp.float32)
    part_sum, cnt_flat = _sc_aggregate(h, src, dst, zacc, zcnt)
    cnt_pair = cnt_flat.reshape(NC, NPAD).T
    out = _tc_combine(h, part_sum, cnt_pair, W, b.reshape(1, D))
    return out


# speedup vs baseline: 1.0866x; 1.0161x over previous
import functools

import jax
import jax.numpy as jnp
from jax import lax
from jax.experimental import pallas as pl
from jax.experimental.pallas import tpu as pltpu
from jax.experimental.pallas import tpu_sc as plsc

N = 10000
E = 320000
D = 128
NC = 2
NS = 16
NW = NC * NS
C = 128
NCHUNK = 80
BF = 4
EPT = C * NCHUNK
EPAD = EPT - E // NW
NPAD = 10112
RPS = NPAD // NS

_mesh = plsc.VectorSubcoreMesh(core_axis_name="c", subcore_axis_name="s")


NHALF = NCHUNK // 2
NPAIR = NHALF // 2


@functools.partial(
    pl.kernel,
    out_type=(
        jax.ShapeDtypeStruct((NC, NPAD, D), jnp.float32),
        jax.ShapeDtypeStruct((NC * NPAD,), jnp.float32),
    ),
    mesh=_mesh,
    scratch_types=[
        pltpu.VMEM((NHALF, C), jnp.int32),
        pltpu.VMEM((NHALF, C), jnp.int32),
        pltpu.VMEM((C, D), jnp.float32),
        pltpu.VMEM((C, D), jnp.float32),
        pltpu.VMEM((C,), jnp.float32),
        pltpu.VMEM_SHARED((NPAD, D), jnp.float32),
        pltpu.VMEM_SHARED((NPAD,), jnp.float32),
        pltpu.SemaphoreType.DMA,
        pltpu.SemaphoreType.DMA,
        pltpu.SemaphoreType.DMA,
        pltpu.SemaphoreType.DMA,
        pltpu.SemaphoreType.DMA,
    ],
)
def _sc_aggregate(h_hbm, src_hbm, dst_hbm, zacc_hbm, zcnt_hbm,
                  sum_hbm, cnt_hbm,
                  src_v, dst_v, rows0_v, rows1_v, ones_v, acc_sp, cnt_sp,
                  semg0, semg1, sems0, sems1, semc):
    c = lax.axis_index("c")
    s = lax.axis_index("s")
    wid = s * NC + c

    for k in range(C // 16):
        ones_v[pl.ds(k * 16, 16)] = jnp.full((16,), 1.0, jnp.float32)

    r0 = s * RPS
    pltpu.async_copy(zacc_hbm, acc_sp.at[pl.ds(r0, RPS)], sems0)

    @pl.when(s == 0)
    def _():
        pltpu.async_copy(zcnt_hbm, cnt_sp, sems1)

    pltpu.make_async_copy(zacc_hbm, acc_sp.at[pl.ds(r0, RPS)], sems0).wait()

    @pl.when(s == 0)
    def _():
        pltpu.make_async_copy(zcnt_hbm, cnt_sp, sems1).wait()

    plsc.subcore_barrier()

    for half in range(2):
        pltpu.async_copy(src_hbm.at[wid, half], src_v, semg0)
        pltpu.async_copy(dst_hbm.at[wid, half], dst_v, semg1)
        pltpu.make_async_copy(src_hbm.at[wid, half], src_v, semg0).wait()
        pltpu.make_async_copy(dst_hbm.at[wid, half], dst_v, semg1).wait()
        pltpu.async_copy(h_hbm.at[src_v.at[0]], rows0_v, semg0)

        def body(g, carry):
            a = 2 * g
            b = a + 1

            @pl.when(g > 0)
            def _():
                pltpu.make_async_copy(rows1_v, acc_sp.at[dst_v.at[b]],
                                      sems1).wait()

            pltpu.async_copy(h_hbm.at[src_v.at[b]], rows1_v, semg1)

            pltpu.make_async_copy(h_hbm.at[src_v.at[a]], rows0_v,
                                  semg0).wait()
            pltpu.async_copy(rows0_v, acc_sp.at[dst_v.at[a]], sems0,
                             add=True)
            pltpu.async_copy(ones_v, cnt_sp.at[dst_v.at[a]], semc, add=True)

            @pl.when(g < NPAIR - 1)
            def _():
                pltpu.make_async_copy(rows0_v, acc_sp.at[dst_v.at[a]],
                                      sems0).wait()
                pltpu.async_copy(h_hbm.at[src_v.at[a + 2]], rows0_v, semg0)

            pltpu.make_async_copy(h_hbm.at[src_v.at[b]], rows1_v,
                                  semg1).wait()
            pltpu.async_copy(rows1_v, acc_sp.at[dst_v.at[b]], sems1,
                             add=True)
            pltpu.async_copy(ones_v, cnt_sp.at[dst_v.at[b]], semc, add=True)
            return carry

        lax.fori_loop(0, NPAIR, body, 0)

        pltpu.make_async_copy(rows0_v, acc_sp.at[dst_v.at[0]], sems0).wait()
        pltpu.make_async_copy(rows1_v, acc_sp.at[dst_v.at[0]], sems1).wait()

        pltpu.make_async_copy(rows0_v.at[pl.ds(0, NHALF * C * 4 // (D * 4))],
                              acc_sp.at[pl.ds(0, NHALF * C * 4 // (D * 4))],
                              semc).wait()

    plsc.subcore_barrier()

    pltpu.sync_copy(acc_sp.at[pl.ds(r0, RPS)], sum_hbm.at[c, pl.ds(r0, RPS)])

    @pl.when(s == 0)
    def _():
        pltpu.sync_copy(cnt_sp, cnt_hbm.at[pl.ds(c * NPAD, NPAD)])


BM = 2000


def _tc_body(h_ref, p_ref, cnt_ref, w_ref, b_ref, o_ref):
    h_b = h_ref[...]
    p = p_ref[0] + p_ref[1]
    cnt = cnt_ref[:, 0:1] + cnt_ref[:, 1:2]
    cmean = p / jnp.maximum(cnt, 1.0)
    z = (
        lax.dot_general(h_b, w_ref[:, :D], (((1,), (1,)), ((), ())),
                        preferred_element_type=jnp.float32)
        + lax.dot_general(cmean, w_ref[:, D:], (((1,), (1,)), ((), ())),
                          preferred_element_type=jnp.float32)
        + b_ref[...]
    )
    nrm = jnp.sqrt(jnp.sum(z * z, axis=1, keepdims=True))
    z = z / jnp.maximum(nrm, 1e-12)
    o_ref[...] = h_b + jnp.maximum(z, 0.0)


def _tc_combine(h, partial, cnts, W, b2):
    grid = (N // BM,)
    return pl.pallas_call(
        _tc_body,
        grid=grid,
        in_specs=[
            pl.BlockSpec((BM, D), lambda i: (i, 0)),
            pl.BlockSpec((NC, BM, D), lambda i: (0, i, 0)),
            pl.BlockSpec((BM, NC), lambda i: (i, 0)),
            pl.BlockSpec((D, 2 * D), lambda i: (0, 0)),
            pl.BlockSpec((1, D), lambda i: (0, 0)),
        ],
        out_specs=pl.BlockSpec((BM, D), lambda i: (i, 0)),
        out_shape=jax.ShapeDtypeStruct((N, D), jnp.float32),
        compiler_params=pltpu.CompilerParams(
            dimension_semantics=("parallel",),
        ),
    )(h, partial, cnts, W, b2)


def _pad_indices(idx, fill):
    per = idx.reshape(NW, E // NW)
    pad = jnp.broadcast_to(fill, (NW, EPAD))
    return jnp.concatenate([per, pad], axis=1).reshape(NW, 2, NCHUNK // 2, C)


def kernel(h, edge_index, W, b):
    src_fill = jnp.arange(EPAD, dtype=jnp.int32) % N
    dst_fill = N + jnp.arange(EPAD, dtype=jnp.int32) % (NPAD - N)
    src = _pad_indices(edge_index[0].astype(jnp.int32), src_fill)
    dst = _pad_indices(edge_index[1].astype(jnp.int32), dst_fill)
    zacc = jnp.zeros((RPS, D), jnp.float32)
    zcnt = jnp.zeros((NPAD,), jnp.float32)
    part_sum, cnt_flat = _sc_aggregate(h, src, dst, zacc, zcnt)
    cnt_pair = cnt_flat.reshape(NC, NPAD).T
    out = _tc_combine(h, part_sum, cnt_pair, W, b.reshape(1, D))
    return out


# speedup vs baseline: 1.0943x; 1.0071x over previous
import functools

import jax
import jax.numpy as jnp
from jax import lax
from jax.experimental import pallas as pl
from jax.experimental.pallas import tpu as pltpu
from jax.experimental.pallas import tpu_sc as plsc

N = 10000
E = 320000
D = 128
NC = 2
NS = 16
NW = NC * NS
C = 128
NCHUNK = 80
BF = 4
EPT = C * NCHUNK
EPAD = EPT - E // NW
NPAD = 10112
RPS = NPAD // NS

_mesh = plsc.VectorSubcoreMesh(core_axis_name="c", subcore_axis_name="s")


NHALF = NCHUNK // 2
NPAIR = NHALF // 2


@functools.partial(
    pl.kernel,
    out_type=(
        jax.ShapeDtypeStruct((NC, NPAD, D), jnp.float32),
        jax.ShapeDtypeStruct((NC * NPAD,), jnp.float32),
    ),
    mesh=_mesh,
    scratch_types=[
        pltpu.VMEM((NHALF, C), jnp.int32),
        pltpu.VMEM((NHALF, C), jnp.int32),
        pltpu.VMEM((C, D), jnp.float32),
        pltpu.VMEM((C, D), jnp.float32),
        pltpu.VMEM((C,), jnp.float32),
        pltpu.VMEM_SHARED((NPAD, D), jnp.float32),
        pltpu.VMEM_SHARED((NPAD,), jnp.float32),
        pltpu.SemaphoreType.DMA,
        pltpu.SemaphoreType.DMA,
        pltpu.SemaphoreType.DMA,
        pltpu.SemaphoreType.DMA,
        pltpu.SemaphoreType.DMA,
    ],
)
def _sc_aggregate(h_hbm, src_hbm, dst_hbm, zacc_hbm, zcnt_hbm,
                  sum_hbm, cnt_hbm,
                  src_v, dst_v, rows0_v, rows1_v, ones_v, acc_sp, cnt_sp,
                  semg0, semg1, sems0, sems1, semc):
    c = lax.axis_index("c")
    s = lax.axis_index("s")
    wid = s * NC + c

    for k in range(C // 16):
        ones_v[pl.ds(k * 16, 16)] = jnp.full((16,), 1.0, jnp.float32)

    r0 = s * RPS
    pltpu.async_copy(zacc_hbm, acc_sp.at[pl.ds(r0, RPS)], sems0)

    @pl.when(s == 0)
    def _():
        pltpu.async_copy(zcnt_hbm, cnt_sp, sems1)

    pltpu.make_async_copy(zacc_hbm, acc_sp.at[pl.ds(r0, RPS)], sems0).wait()

    @pl.when(s == 0)
    def _():
        pltpu.make_async_copy(zcnt_hbm, cnt_sp, sems1).wait()

    plsc.subcore_barrier()

    for half in range(2):
        pltpu.async_copy(src_hbm.at[wid, half], src_v, semg0)
        pltpu.async_copy(dst_hbm.at[wid, half], dst_v, semg1)
        pltpu.make_async_copy(src_hbm.at[wid, half], src_v, semg0).wait()
        pltpu.make_async_copy(dst_hbm.at[wid, half], dst_v, semg1).wait()
        pltpu.async_copy(h_hbm.at[src_v.at[0]], rows0_v, semg0)

        def body(g, carry):
            a = 2 * g
            b = a + 1

            @pl.when(g > 0)
            def _():
                pltpu.make_async_copy(rows1_v, acc_sp.at[dst_v.at[b]],
                                      sems1).wait()

            pltpu.async_copy(h_hbm.at[src_v.at[b]], rows1_v, semg1)

            pltpu.make_async_copy(h_hbm.at[src_v.at[a]], rows0_v,
                                  semg0).wait()
            pltpu.async_copy(rows0_v, acc_sp.at[dst_v.at[a]], sems0,
                             add=True)
            pltpu.async_copy(ones_v, cnt_sp.at[dst_v.at[a]], semc, add=True)

            @pl.when(g < NPAIR - 1)
            def _():
                pltpu.make_async_copy(rows0_v, acc_sp.at[dst_v.at[a]],
                                      sems0).wait()
                pltpu.async_copy(h_hbm.at[src_v.at[a + 2]], rows0_v, semg0)

            pltpu.make_async_copy(h_hbm.at[src_v.at[b]], rows1_v,
                                  semg1).wait()
            pltpu.async_copy(rows1_v, acc_sp.at[dst_v.at[b]], sems1,
                             add=True)
            pltpu.async_copy(ones_v, cnt_sp.at[dst_v.at[b]], semc, add=True)
            return carry

        lax.fori_loop(0, NPAIR, body, 0)

        pltpu.make_async_copy(rows0_v, acc_sp.at[dst_v.at[0]], sems0).wait()
        pltpu.make_async_copy(rows1_v, acc_sp.at[dst_v.at[0]], sems1).wait()

        pltpu.make_async_copy(rows0_v.at[pl.ds(0, NHALF * C * 4 // (D * 4))],
                              acc_sp.at[pl.ds(0, NHALF * C * 4 // (D * 4))],
                              semc).wait()

    plsc.subcore_barrier()

    pltpu.sync_copy(acc_sp.at[pl.ds(r0, RPS)], sum_hbm.at[c, pl.ds(r0, RPS)])

    @pl.when(s == 0)
    def _():
        pltpu.sync_copy(cnt_sp, cnt_hbm.at[pl.ds(c * NPAD, NPAD)])


BM = 5000


def _tc_body(h_ref, p_ref, cnt_ref, w_ref, b_ref, o_ref):
    h_b = h_ref[...]
    p = p_ref[0] + p_ref[1]
    cnt = cnt_ref[:, 0:1] + cnt_ref[:, 1:2]
    cmean = p / jnp.maximum(cnt, 1.0)
    z = (
        lax.dot_general(h_b, w_ref[:, :D], (((1,), (1,)), ((), ())),
                        preferred_element_type=jnp.float32)
        + lax.dot_general(cmean, w_ref[:, D:], (((1,), (1,)), ((), ())),
                          preferred_element_type=jnp.float32)
        + b_ref[...]
    )
    nrm = jnp.sqrt(jnp.sum(z * z, axis=1, keepdims=True))
    z = z / jnp.maximum(nrm, 1e-12)
    o_ref[...] = h_b + jnp.maximum(z, 0.0)


def _tc_combine(h, partial, cnts, W, b2):
    grid = (N // BM,)
    return pl.pallas_call(
        _tc_body,
        grid=grid,
        in_specs=[
            pl.BlockSpec((BM, D), lambda i: (i, 0)),
            pl.BlockSpec((NC, BM, D), lambda i: (0, i, 0)),
            pl.BlockSpec((BM, NC), lambda i: (i, 0)),
            pl.BlockSpec((D, 2 * D), lambda i: (0, 0)),
            pl.BlockSpec((1, D), lambda i: (0, 0)),
        ],
        out_specs=pl.BlockSpec((BM, D), lambda i: (i, 0)),
        out_shape=jax.ShapeDtypeStruct((N, D), jnp.float32),
        compiler_params=pltpu.CompilerParams(
            dimension_semantics=("parallel",),
        ),
    )(h, partial, cnts, W, b2)


def _pad_indices(idx, fill):
    per = idx.reshape(NW, E // NW)
    pad = jnp.broadcast_to(fill, (NW, EPAD))
    return jnp.concatenate([per, pad], axis=1).reshape(NW, 2, NCHUNK // 2, C)


def kernel(h, edge_index, W, b):
    src_fill = jnp.arange(EPAD, dtype=jnp.int32) % N
    dst_fill = N + jnp.arange(EPAD, dtype=jnp.int32) % (NPAD - N)
    src = _pad_indices(edge_index[0].astype(jnp.int32), src_fill)
    dst = _pad_indices(edge_index[1].astype(jnp.int32), dst_fill)
    zacc = jnp.zeros((RPS, D), jnp.float32)
    zcnt = jnp.zeros((NPAD,), jnp.float32)
    part_sum, cnt_flat = _sc_aggregate(h, src, dst, zacc, zcnt)
    cnt_pair = cnt_flat.reshape(NC, NPAD).T
    out = _tc_combine(h, part_sum, cnt_pair, W, b.reshape(1, D))
    return out


# speedup vs baseline: 1.0954x; 1.0010x over previous
import functools

import jax
import jax.numpy as jnp
from jax import lax
from jax.experimental import pallas as pl
from jax.experimental.pallas import tpu as pltpu
from jax.experimental.pallas import tpu_sc as plsc

N = 10000
E = 320000
D = 128
NC = 2
NS = 16
NW = NC * NS
C = 128
NCHUNK = 80
EPT = C * NCHUNK
EPAD = EPT - E // NW
NPAD = 10112
RPS = NPAD // NS

_mesh = plsc.VectorSubcoreMesh(core_axis_name="c", subcore_axis_name="s")


NHALF = NCHUNK // 2
NPAIR = NHALF // 2


@functools.partial(
    pl.kernel,
    out_type=(
        jax.ShapeDtypeStruct((NC, NPAD, D), jnp.float32),
        jax.ShapeDtypeStruct((NC * NPAD,), jnp.float32),
    ),
    mesh=_mesh,
    scratch_types=[
        pltpu.VMEM((NHALF, C), jnp.int32),
        pltpu.VMEM((NHALF, C), jnp.int32),
        pltpu.VMEM((C, D), jnp.float32),
        pltpu.VMEM((C, D), jnp.float32),
        pltpu.VMEM((C,), jnp.float32),
        pltpu.VMEM_SHARED((NPAD, D), jnp.float32),
        pltpu.VMEM_SHARED((NPAD,), jnp.float32),
        pltpu.SemaphoreType.DMA,
        pltpu.SemaphoreType.DMA,
        pltpu.SemaphoreType.DMA,
        pltpu.SemaphoreType.DMA,
        pltpu.SemaphoreType.DMA,
    ],
)
def _sc_aggregate(h_hbm, src_hbm, dst_hbm, zacc_hbm, zcnt_hbm,
                  sum_hbm, cnt_hbm,
                  src_v, dst_v, rows0_v, rows1_v, ones_v, acc_sp, cnt_sp,
                  semg0, semg1, sems0, sems1, semc):
    c = lax.axis_index("c")
    s = lax.axis_index("s")
    wid = s * NC + c

    for k in range(C // 16):
        ones_v[pl.ds(k * 16, 16)] = jnp.full((16,), 1.0, jnp.float32)

    r0 = s * RPS
    pltpu.async_copy(zacc_hbm, acc_sp.at[pl.ds(r0, RPS)], sems0)

    @pl.when(s == 0)
    def _():
        pltpu.async_copy(zcnt_hbm, cnt_sp, sems1)

    pltpu.make_async_copy(zacc_hbm, acc_sp.at[pl.ds(r0, RPS)], sems0).wait()

    @pl.when(s == 0)
    def _():
        pltpu.make_async_copy(zcnt_hbm, cnt_sp, sems1).wait()

    plsc.subcore_barrier()

    for half in range(2):
        pltpu.async_copy(src_hbm.at[wid, half], src_v, semg0)
        pltpu.async_copy(dst_hbm.at[wid, half], dst_v, semg1)
        pltpu.make_async_copy(src_hbm.at[wid, half], src_v, semg0).wait()
        pltpu.make_async_copy(dst_hbm.at[wid, half], dst_v, semg1).wait()
        pltpu.async_copy(h_hbm.at[src_v.at[0]], rows0_v, semg0)

        def body(g, carry):
            a = 2 * g
            b = a + 1

            @pl.when(g > 0)
            def _():
                pltpu.make_async_copy(rows1_v, acc_sp.at[dst_v.at[b]],
                                      sems1).wait()

            pltpu.async_copy(h_hbm.at[src_v.at[b]], rows1_v, semg1)

            pltpu.make_async_copy(h_hbm.at[src_v.at[a]], rows0_v,
                                  semg0).wait()
            pltpu.async_copy(rows0_v, acc_sp.at[dst_v.at[a]], sems0,
                             add=True)
            pltpu.async_copy(ones_v, cnt_sp.at[dst_v.at[a]], semc, add=True)

            @pl.when(g < NPAIR - 1)
            def _():
                pltpu.make_async_copy(rows0_v, acc_sp.at[dst_v.at[a]],
                                      sems0).wait()
                pltpu.async_copy(h_hbm.at[src_v.at[a + 2]], rows0_v, semg0)

            pltpu.make_async_copy(h_hbm.at[src_v.at[b]], rows1_v,
                                  semg1).wait()
            pltpu.async_copy(rows1_v, acc_sp.at[dst_v.at[b]], sems1,
                             add=True)
            pltpu.async_copy(ones_v, cnt_sp.at[dst_v.at[b]], semc, add=True)
            return carry

        lax.fori_loop(0, NPAIR, body, 0)

        pltpu.make_async_copy(rows0_v, acc_sp.at[dst_v.at[0]], sems0).wait()
        pltpu.make_async_copy(rows1_v, acc_sp.at[dst_v.at[0]], sems1).wait()

        pltpu.make_async_copy(rows0_v.at[pl.ds(0, NHALF * C * 4 // (D * 4))],
                              acc_sp.at[pl.ds(0, NHALF * C * 4 // (D * 4))],
                              semc).wait()

    plsc.subcore_barrier()

    pltpu.sync_copy(acc_sp.at[pl.ds(r0, RPS)], sum_hbm.at[c, pl.ds(r0, RPS)])

    @pl.when(s == 0)
    def _():
        pltpu.sync_copy(cnt_sp, cnt_hbm.at[pl.ds(c * NPAD, NPAD)])


BM = 5000


def _tc_body(h_ref, p_ref, cnt_ref, w_ref, b_ref, o_ref):
    h_b = h_ref[...]
    p = p_ref[0] + p_ref[1]
    cnt = cnt_ref[:, 0:1] + cnt_ref[:, 1:2]
    cmean = p / jnp.maximum(cnt, 1.0)
    z = (
        lax.dot_general(h_b, w_ref[:, :D], (((1,), (1,)), ((), ())),
                        preferred_element_type=jnp.float32)
        + lax.dot_general(cmean, w_ref[:, D:], (((1,), (1,)), ((), ())),
                          preferred_element_type=jnp.float32)
        + b_ref[...]
    )
    nrm = jnp.sqrt(jnp.sum(z * z, axis=1, keepdims=True))
    z = z / jnp.maximum(nrm, 1e-12)
    o_ref[...] = h_b + jnp.maximum(z, 0.0)


def _tc_combine(h, partial, cnts, W, b2):
    grid = (N // BM,)
    return pl.pallas_call(
        _tc_body,
        grid=grid,
        in_specs=[
            pl.BlockSpec((BM, D), lambda i: (i, 0)),
            pl.BlockSpec((NC, BM, D), lambda i: (0, i, 0)),
            pl.BlockSpec((BM, NC), lambda i: (i, 0)),
            pl.BlockSpec((D, 2 * D), lambda i: (0, 0)),
            pl.BlockSpec((1, D), lambda i: (0, 0)),
        ],
        out_specs=pl.BlockSpec((BM, D), lambda i: (i, 0)),
        out_shape=jax.ShapeDtypeStruct((N, D), jnp.float32),
        compiler_params=pltpu.CompilerParams(
            dimension_semantics=("parallel",),
        ),
    )(h, partial, cnts, W, b2)


def _pad_indices(idx, fill):
    per = idx.reshape(NW, E // NW)
    pad = jnp.broadcast_to(fill, (NW, EPAD))
    return jnp.concatenate([per, pad], axis=1).reshape(NW, 2, NCHUNK // 2, C)


def kernel(h, edge_index, W, b):
    src_fill = jnp.arange(EPAD, dtype=jnp.int32) % N
    dst_fill = N + jnp.arange(EPAD, dtype=jnp.int32) % (NPAD - N)
    src = _pad_indices(edge_index[0].astype(jnp.int32), src_fill)
    dst = _pad_indices(edge_index[1].astype(jnp.int32), dst_fill)
    zacc = jnp.zeros((RPS, D), jnp.float32)
    zcnt = jnp.zeros((NPAD,), jnp.float32)
    part_sum, cnt_flat = _sc_aggregate(h, src, dst, zacc, zcnt)
    cnt_pair = cnt_flat.reshape(NC, NPAD).T
    out = _tc_combine(h, part_sum, cnt_pair, W, b.reshape(1, D))
    return out
